# Initial kernel scaffold; baseline (speedup 1.0000x reference)
import jax
import jax.numpy as jnp
from jax.experimental import pallas as pl


def kernel(x, embed, W, b):
    raise NotImplementedError("write your pallas kernel here")


# speedup vs baseline: 1.7185x; 1.7185x over previous
import functools

import jax
import jax.numpy as jnp
from jax import lax
from jax.experimental import pallas as pl
from jax.experimental.pallas import tpu as pltpu
from jax.experimental.pallas import tpu_sc as plsc

_VOCAB = 1000000
_EMB = 16
_CLS = 2

_ROWS_PER_BLK = 10000


def _project_body(e_ref, w_ref, b_ref, o_ref):
    acc = lax.dot_general(
        e_ref[...], w_ref[...],
        dimension_numbers=(((1,), (1,)), ((), ())),
        preferred_element_type=jnp.float32,
    )
    o_ref[...] = acc + b_ref[...]


def _project_table(embed, W, b2d):
    grid = _VOCAB // _ROWS_PER_BLK
    return pl.pallas_call(
        _project_body,
        grid=(grid,),
        in_specs=[
            pl.BlockSpec((_ROWS_PER_BLK, _EMB), lambda i: (i, 0)),
            pl.BlockSpec((_CLS, _EMB), lambda i: (0, 0)),
            pl.BlockSpec((1, _CLS), lambda i: (0, 0)),
        ],
        out_specs=pl.BlockSpec((_ROWS_PER_BLK, _CLS), lambda i: (i, 0)),
        out_shape=jax.ShapeDtypeStruct((_VOCAB, _CLS), jnp.float32),
    )(embed, W, b2d)


_IDX_W = 128
_NSTREAM = 16


def _make_gather(n_flat):
    info = plsc.get_sparse_core_info()
    nc, ns = info.num_cores, info.num_subcores
    nw = nc * ns
    v_flat = _VOCAB * _CLS
    v_per_s = v_flat // ns
    n_rows = n_flat // _IDX_W
    per_w = n_rows // nw
    steps = per_w // _NSTREAM
    mesh = plsc.VectorSubcoreMesh(core_axis_name="c", subcore_axis_name="s")

    @functools.partial(
        pl.kernel,
        out_type=jax.ShapeDtypeStruct((n_rows, _IDX_W), jnp.float32),
        mesh=mesh,
        scratch_types=[
            pltpu.VMEM_SHARED((v_flat,), jnp.float32),
            pltpu.VMEM((_NSTREAM, _IDX_W), jnp.int32),
            pltpu.VMEM((_NSTREAM, _IDX_W), jnp.float32),
            pltpu.SemaphoreType.DMA,
        ],
        compiler_params=pltpu.CompilerParams(use_tc_tiling_on_sc=False),
    )
    def gather_kernel(table_hbm, idx_hbm, out_hbm, table_sh, idx_v, rows_v, sem):
        cid = lax.axis_index("c")
        sid = lax.axis_index("s")
        wid = sid * nc + cid
        base = wid * per_w
        pltpu.sync_copy(table_hbm.at[pl.ds(sid * v_per_s, v_per_s)],
                        table_sh.at[pl.ds(sid * v_per_s, v_per_s)])
        plsc.subcore_barrier()

        def step(j, carry):
            off = base + j * _NSTREAM
            pltpu.sync_copy(idx_hbm.at[pl.ds(off, _NSTREAM)], idx_v)
            copies = [
                pltpu.async_copy(table_sh.at[idx_v.at[k]], rows_v.at[k], sem)
                for k in range(_NSTREAM)
            ]
            for c in copies:
                c.wait()
            pltpu.sync_copy(rows_v, out_hbm.at[pl.ds(off, _NSTREAM)])
            return carry

        lax.fori_loop(0, steps, step, 0)

    return gather_kernel


def kernel(x, embed, W, b):
    B, L = x.shape
    table = _project_table(embed, W, b.reshape(1, _CLS)).reshape(-1)
    n_flat = B * L * _CLS
    idx2 = (x.reshape(-1, 1) * _CLS
            + jnp.arange(_CLS, dtype=jnp.int32)).reshape(-1, _IDX_W)
    out = _make_gather(n_flat)(table, idx2)
    return out.reshape(B, L, _CLS)
